# baseline (speedup 1.0000x reference)
import functools

import jax
import jax.numpy as jnp
from jax import lax
from jax.experimental import pallas as pl
from jax.experimental.pallas import tpu as pltpu
from jax.experimental.pallas import tpu_sc as plsc

_ROWS = 1_000_000
_D = 32
_NW = 32
_CHUNK = 1008
_NCHUNK = 31
_WROWS = _CHUNK * _NCHUNK
_TAIL = _ROWS - _NW * _WROWS
_GROUPS = _CHUNK // 16


def _scan_sc(q_hbm, x_hbm, outv_hbm, outi_hbm, qv, xbuf, stgv, stgi, sems, qsem):
    w = lax.axis_index("s") * 2 + lax.axis_index("c")
    row0 = w * _WROWS

    pltpu.async_copy(q_hbm.at[0], qv, qsem).wait()
    qa = qv[pl.ds(0, 16)]
    qb = qv[pl.ds(16, 16)]
    iota = lax.iota(jnp.int32, 16)

    pltpu.make_async_copy(
        x_hbm.at[pl.ds(row0, _CHUNK)], xbuf.at[0], sems.at[0]).start()

    def scan_rows(sel, base, nrows, carry):
        bufv = jnp.full((16,), sel, jnp.int32)

        def group_body(g, carry2):
            minv, mini = carry2
            rows = g * 16 + iota
            acc = jnp.zeros((16,), jnp.float32)
            for f in range(_D):
                col = jnp.full((16,), f, jnp.int32)
                v = plsc.load_gather(xbuf, [bufv, rows, col])
                qf = qa[f] if f < 16 else qb[f - 16]
                t = v - qf
                acc = acc + t * t
            rid = (base + g * 16) + iota
            cond = acc < minv
            minv = jnp.where(cond, acc, minv)
            mini = jnp.where(cond, rid, mini)
            return minv, mini

        return lax.fori_loop(0, nrows // 16, group_body, carry)

    def chunk_body(c, carry):
        sel = lax.rem(c, 2)
        nxt = lax.rem(c + 1, 2)

        @pl.when(c + 1 < _NCHUNK)
        def _():
            pltpu.make_async_copy(
                x_hbm.at[pl.ds(row0 + (c + 1) * _CHUNK, _CHUNK)],
                xbuf.at[nxt], sems.at[nxt]).start()

        pltpu.make_async_copy(
            x_hbm.at[pl.ds(row0 + c * _CHUNK, _CHUNK)],
            xbuf.at[sel], sems.at[sel]).wait()

        return scan_rows(sel, row0 + c * _CHUNK, _CHUNK, carry)

    init = (jnp.full((16,), jnp.inf, jnp.float32), jnp.zeros((16,), jnp.int32))
    minv, mini = lax.fori_loop(0, _NCHUNK, chunk_body, init)

    @pl.when(w == _NW - 1)
    def _():
        tbase = _NW * _WROWS
        tcp = pltpu.make_async_copy(
            x_hbm.at[pl.ds(tbase, _TAIL)], xbuf.at[0, pl.ds(0, _TAIL)],
            sems.at[0])
        tcp.start()
        tcp.wait()

    minv, mini = lax.cond(
        w == _NW - 1,
        lambda cr: scan_rows(jnp.int32(0), _NW * _WROWS, _TAIL, cr),
        lambda cr: cr,
        (minv, mini))

    stgv[...] = minv
    stgi[...] = mini
    pltpu.async_copy(stgv, outv_hbm.at[w], sems.at[0]).wait()
    pltpu.async_copy(stgi, outi_hbm.at[w], sems.at[0]).wait()


def _merge_tc(outv_ref, outi_ref, tt_ref, out_ref, sem):
    v = outv_ref[...]
    mi = outi_ref[...]
    m = jnp.min(v)
    best = jnp.min(jnp.where(v == m, mi, jnp.int32(2**30)))
    cp = pltpu.make_async_copy(tt_ref.at[pl.ds(best, 1)], out_ref, sem)
    cp.start()
    cp.wait()


@jax.jit
def kernel(in_vel, train_obs_vel, train_target_vel):
    mesh = plsc.VectorSubcoreMesh(core_axis_name="c", subcore_axis_name="s")

    cp = pltpu.CompilerParams(
        needs_layout_passes=False, use_tc_tiling_on_sc=False)

    scan = functools.partial(
        pl.kernel,
        mesh=mesh,
        compiler_params=cp,
        out_type=[
            jax.ShapeDtypeStruct((_NW, 16), jnp.float32),
            jax.ShapeDtypeStruct((_NW, 16), jnp.int32),
        ],
        scratch_types=[
            pltpu.VMEM((32,), jnp.float32),
            pltpu.VMEM((2, _CHUNK, _D), jnp.float32),
            pltpu.VMEM((16,), jnp.float32),
            pltpu.VMEM((16,), jnp.int32),
            pltpu.SemaphoreType.DMA((2,)),
            pltpu.SemaphoreType.DMA,
        ],
    )(_scan_sc)

    outv, outi = scan(in_vel, train_obs_vel)

    out = pl.pallas_call(
        _merge_tc,
        in_specs=[
            pl.BlockSpec((_NW, 16), lambda: (0, 0)),
            pl.BlockSpec((_NW, 16), lambda: (0, 0)),
            pl.BlockSpec(memory_space=pl.MemorySpace.ANY),
        ],
        out_specs=pl.BlockSpec((1, _D), lambda: (0, 0)),
        out_shape=jax.ShapeDtypeStruct((1, _D), jnp.float32),
        scratch_shapes=[pltpu.SemaphoreType.DMA],
    )(outv, outi, train_target_vel)
    return out[0]


# speedup vs baseline: 1.2258x; 1.2258x over previous
import jax
import jax.numpy as jnp
from jax.experimental import pallas as pl
from jax.experimental.pallas import tpu as pltpu

_ROWS = 1_000_000
_D = 32
_B = 20000
_G = _ROWS // _B


def _scan_tc(q_ref, x_ref, tt_ref, out_ref, best_ref, besti_ref, sem):
    i = pl.program_id(0)

    @pl.when(i == 0)
    def _():
        best_ref[0] = jnp.inf
        besti_ref[0] = jnp.int32(0)

    x = x_ref[...]
    t = x - q_ref[...]
    s = jnp.sum(t * t, axis=1)
    m = jnp.min(s)
    ai = jnp.argmin(s).astype(jnp.int32)
    cond = m < best_ref[0]
    besti_ref[0] = jnp.where(cond, i * _B + ai, besti_ref[0])
    best_ref[0] = jnp.where(cond, m, best_ref[0])

    @pl.when(i == _G - 1)
    def _():
        cp = pltpu.make_async_copy(
            tt_ref.at[pl.ds(besti_ref[0], 1)], out_ref, sem)
        cp.start()
        cp.wait()


@jax.jit
def kernel(in_vel, train_obs_vel, train_target_vel):
    out = pl.pallas_call(
        _scan_tc,
        grid=(_G,),
        in_specs=[
            pl.BlockSpec((1, _D), lambda i: (0, 0)),
            pl.BlockSpec((_B, _D), lambda i: (i, 0)),
            pl.BlockSpec(memory_space=pl.MemorySpace.ANY),
        ],
        out_specs=pl.BlockSpec((1, _D), lambda i: (0, 0)),
        out_shape=jax.ShapeDtypeStruct((1, _D), jnp.float32),
        scratch_shapes=[
            pltpu.SMEM((1,), jnp.float32),
            pltpu.SMEM((1,), jnp.int32),
            pltpu.SemaphoreType.DMA,
        ],
        compiler_params=pltpu.CompilerParams(
            dimension_semantics=("arbitrary",),
        ),
    )(in_vel, train_obs_vel, train_target_vel)
    return out[0]
